# baseline (speedup 1.0000x reference)
import functools

import jax
import jax.numpy as jnp
from jax import lax
from jax.experimental import pallas as pl
from jax.experimental.pallas import tpu as pltpu
from jax.experimental.pallas import tpu_sc as plsc

_MAX_NORM = 10.0


_COLS = 512
_HALF = 50176


def _scale_body(ta_ref, tb_ref, o_ref):
    def scaled_t(x):
        ss = jnp.sum(x * x, axis=0, keepdims=True)
        s = jnp.minimum(1.0, _MAX_NORM * lax.rsqrt(jnp.maximum(ss, 1e-24)))
        return (x * s).T
    o_ref[...] = jnp.concatenate(
        [scaled_t(ta_ref[...]), scaled_t(tb_ref[...])], axis=1)


def _scale_table_t(table_t):
    d, v = table_t.shape
    nblk = _HALF // _COLS
    return pl.pallas_call(
        _scale_body,
        grid=(nblk,),
        in_specs=[
            pl.BlockSpec((d, _COLS), lambda i: (0, i)),
            pl.BlockSpec((d, _COLS), lambda i: (0, i + nblk)),
        ],
        out_specs=pl.BlockSpec((_COLS, 2 * d), lambda i: (i, 0)),
        out_shape=jax.ShapeDtypeStruct((_HALF, 2 * d), table_t.dtype),
    )(table_t, table_t)


_WINDOW = 128


def _sc_gather(table, idx2d):
    n = idx2d.shape[1]
    d = table.shape[1]
    assert n % (_WINDOW * 32) == 0
    mesh = plsc.VectorSubcoreMesh(core_axis_name="core",
                                  subcore_axis_name="subcore")

    @functools.partial(
        pl.kernel,
        out_type=jax.ShapeDtypeStruct((n, d), table.dtype),
        mesh=mesh,
        compiler_params=pltpu.CompilerParams(use_tc_tiling_on_sc=False),
    )
    def k(x_hbm, i_hbm, o_hbm):
        def body(i_vmem, o_vmem):
            pltpu.sync_copy(x_hbm.at[i_vmem.at[0]], o_vmem)

        pltpu.emit_pipeline(
            body,
            grid=(n // _WINDOW,),
            in_specs=[pl.BlockSpec((1, _WINDOW), index_map=lambda i: (0, i))],
            out_specs=[pl.BlockSpec((_WINDOW, d), index_map=lambda i: (i, 0))],
            core_axis_name=("core", "subcore"),
            dimension_semantics=(pltpu.PARALLEL,),
        )(i_hbm, o_hbm)

    return k(table, idx2d)


def _fin_body(g_ref, o_ref):
    x = g_ref[...]
    v = jnp.concatenate([x[:, :64].T, x[:, 64:].T], axis=1)
    o_ref[...] = v[None]


def _transpose_fin(g2, s, b):
    d = 64
    return pl.pallas_call(
        _fin_body,
        grid=(s,),
        in_specs=[pl.BlockSpec((b // 2, 2 * d), lambda i: (i, 0))],
        out_specs=pl.BlockSpec((1, d, b), lambda i: (i, 0, 0)),
        out_shape=jax.ShapeDtypeStruct((s, d, b), jnp.float32),
    )(g2)


def kernel(indices, table):
    v, d = table.shape
    b, s = indices.shape
    scaled = _scale_table_t(table.T).reshape(2 * _HALF, d)
    idx = indices.T.reshape(s, 2, b // 2).transpose(0, 2, 1)
    idx = idx.reshape(1, -1).astype(jnp.int32)
    idx = 2 * idx - jnp.where(idx < _HALF, 0, 2 * _HALF - 1)
    g = _sc_gather(scaled, idx)
    out_t = _transpose_fin(g.reshape(b * s // 2, 2 * d), s, b)
    return out_t.transpose(2, 0, 1)


# speedup vs baseline: 1.3430x; 1.3430x over previous
import functools

import jax
import jax.numpy as jnp
from jax import lax
from jax.experimental import pallas as pl
from jax.experimental.pallas import tpu as pltpu
from jax.experimental.pallas import tpu_sc as plsc

_MAX_NORM = 10.0


_COLS = 1024
_HALF = 50176


def _scale_body(ta_ref, tb_ref, o_ref):
    def scaled_t(x):
        ss = jnp.sum(x * x, axis=0, keepdims=True)
        s = jnp.minimum(1.0, _MAX_NORM * lax.rsqrt(jnp.maximum(ss, 1e-24)))
        return (x * s).T
    o_ref[...] = jnp.concatenate(
        [scaled_t(ta_ref[...]), scaled_t(tb_ref[...])], axis=1)


def _scale_table_t(table_t):
    d, v = table_t.shape
    nblk = _HALF // _COLS
    return pl.pallas_call(
        _scale_body,
        grid=(nblk,),
        in_specs=[
            pl.BlockSpec((d, _COLS), lambda i: (0, i)),
            pl.BlockSpec((d, _COLS), lambda i: (0, i + nblk)),
        ],
        out_specs=pl.BlockSpec((_COLS, 2 * d), lambda i: (i, 0)),
        out_shape=jax.ShapeDtypeStruct((_HALF, 2 * d), table_t.dtype),
    )(table_t, table_t)


_WINDOW = 128


def _sc_gather(table, idx2d):
    n = idx2d.shape[1]
    d = table.shape[1]
    assert n % (_WINDOW * 32) == 0
    mesh = plsc.VectorSubcoreMesh(core_axis_name="core",
                                  subcore_axis_name="subcore")

    @functools.partial(
        pl.kernel,
        out_type=jax.ShapeDtypeStruct((n, d), table.dtype),
        mesh=mesh,
        compiler_params=pltpu.CompilerParams(use_tc_tiling_on_sc=False),
    )
    def k(x_hbm, i_hbm, o_hbm):
        def body(i_vmem, o_vmem):
            pltpu.sync_copy(x_hbm.at[i_vmem.at[0]], o_vmem)

        pltpu.emit_pipeline(
            body,
            grid=(n // _WINDOW,),
            in_specs=[pl.BlockSpec((1, _WINDOW), index_map=lambda i: (0, i))],
            out_specs=[pl.BlockSpec((_WINDOW, d), index_map=lambda i: (i, 0))],
            core_axis_name=("core", "subcore"),
            dimension_semantics=(pltpu.PARALLEL,),
        )(i_hbm, o_hbm)

    return k(table, idx2d)


def _idx_body(i_ref, o_ref):
    x = i_ref[...]
    x = 2 * x - jnp.where(x < _HALF, 0, 2 * _HALF - 1)
    b = x.shape[1]
    xt = x.T
    z = jnp.stack([xt[: b // 2], xt[b // 2:]], axis=1)
    y = z.reshape(b, 8).T
    rows, cols = o_ref.shape
    o_ref[...] = y.reshape(8, rows // 8, cols).reshape(rows, cols)


def _idx_prep(idx_t):
    s, b = idx_t.shape
    n = s * b
    grid = (s + 7) // 8
    return pl.pallas_call(
        _idx_body,
        grid=(grid,),
        in_specs=[pl.BlockSpec((8, b), lambda i: (i, 0))],
        out_specs=pl.BlockSpec((8 * b // 128, 128), lambda i: (i, 0)),
        out_shape=jax.ShapeDtypeStruct((n // 128, 128), jnp.int32),
    )(idx_t)


def _fin_body(g_ref, o_ref):
    x = g_ref[...]
    v = jnp.concatenate([x[:, :64].T, x[:, 64:].T], axis=1)
    o_ref[...] = v[None]


def _transpose_fin(g2, s, b):
    d = 64
    return pl.pallas_call(
        _fin_body,
        grid=(s,),
        in_specs=[pl.BlockSpec((b // 2, 2 * d), lambda i: (i, 0))],
        out_specs=pl.BlockSpec((1, d, b), lambda i: (i, 0, 0)),
        out_shape=jax.ShapeDtypeStruct((s, d, b), jnp.float32),
    )(g2)


def kernel(indices, table):
    v, d = table.shape
    b, s = indices.shape
    scaled = _scale_table_t(table.T).reshape(2 * _HALF, d)
    idx = _idx_prep(indices.T.astype(jnp.int32)).reshape(1, b * s)
    g = _sc_gather(scaled, idx)
    out_t = _transpose_fin(g.reshape(b * s // 2, 2 * d), s, b)
    return out_t.transpose(2, 0, 1)


# speedup vs baseline: 1.5575x; 1.1597x over previous
import functools

import jax
import jax.numpy as jnp
from jax import lax
from jax.experimental import pallas as pl
from jax.experimental.pallas import tpu as pltpu
from jax.experimental.pallas import tpu_sc as plsc

_MAX_NORM = 10.0


_COLS = 1536
_HALF = 50688


def _scale_body(ta_ref, tb_ref, o_ref):
    def scaled_t(x):
        ss = jnp.sum(x * x, axis=0, keepdims=True)
        s = jnp.minimum(1.0, _MAX_NORM * lax.rsqrt(jnp.maximum(ss, 1e-24)))
        return (x * s).T
    o_ref[...] = jnp.concatenate(
        [scaled_t(ta_ref[...]), scaled_t(tb_ref[...])], axis=1)


def _scale_table_t(table_t):
    d, v = table_t.shape
    nblk = _HALF // _COLS
    return pl.pallas_call(
        _scale_body,
        grid=(nblk,),
        in_specs=[
            pl.BlockSpec((d, _COLS), lambda i: (0, i)),
            pl.BlockSpec((d, _COLS), lambda i: (0, i + nblk)),
        ],
        out_specs=pl.BlockSpec((_COLS, 2 * d), lambda i: (i, 0)),
        out_shape=jax.ShapeDtypeStruct((_HALF, 2 * d), table_t.dtype),
    )(table_t, table_t)


_WINDOW = 128


def _sc_gather(table, idx2d, w0, nw):
    d = table.shape[1]
    assert nw % 32 == 0
    mesh = plsc.VectorSubcoreMesh(core_axis_name="core",
                                  subcore_axis_name="subcore")

    @functools.partial(
        pl.kernel,
        out_type=jax.ShapeDtypeStruct((nw * _WINDOW, d), table.dtype),
        mesh=mesh,
        compiler_params=pltpu.CompilerParams(use_tc_tiling_on_sc=False),
    )
    def k(x_hbm, i_hbm, o_hbm):
        def body(i_vmem, o_vmem):
            pltpu.sync_copy(x_hbm.at[i_vmem.at[0]], o_vmem)

        pltpu.emit_pipeline(
            body,
            grid=(nw,),
            in_specs=[pl.BlockSpec((1, _WINDOW),
                                   index_map=lambda i: (0, i + w0))],
            out_specs=[pl.BlockSpec((_WINDOW, d), index_map=lambda i: (i, 0))],
            core_axis_name=("core", "subcore"),
            dimension_semantics=(pltpu.PARALLEL,),
        )(i_hbm, o_hbm)

    return k(table, idx2d)


def _idx_body(i_ref, o_ref):
    x = i_ref[...]
    x = 2 * x - jnp.where(x < _HALF, 0, 2 * _HALF - 1)
    b = x.shape[1]
    xt = x.T
    z = jnp.stack([xt[: b // 2], xt[b // 2:]], axis=1)
    y = z.reshape(b, 8).T
    rows, cols = o_ref.shape
    o_ref[...] = y.reshape(8, rows // 8, cols).reshape(rows, cols)


def _idx_prep(idx_t):
    s, b = idx_t.shape
    n = s * b
    grid = (s + 7) // 8
    return pl.pallas_call(
        _idx_body,
        grid=(grid,),
        in_specs=[pl.BlockSpec((8, b), lambda i: (i, 0))],
        out_specs=pl.BlockSpec((8 * b // 128, 128), lambda i: (i, 0)),
        out_shape=jax.ShapeDtypeStruct((n // 128, 128), jnp.int32),
    )(idx_t)


def _fin_body(g_ref, *rest):
    o_ref = rest[-1]
    x = g_ref[...]
    v = jnp.concatenate([x[:, :64].T, x[:, 64:].T], axis=1)
    o_ref[...] = v[None]


def _transpose_fin(g2c, s_total, s0, ns, b, prev=None):
    d = 64
    in_specs = [pl.BlockSpec((b // 2, 2 * d), lambda i: (i, 0))]
    args = [g2c]
    kwargs = {}
    if prev is not None:
        in_specs.append(pl.BlockSpec(memory_space=pl.ANY))
        args.append(prev)
        kwargs["input_output_aliases"] = {1: 0}
    return pl.pallas_call(
        _fin_body,
        grid=(ns,),
        in_specs=in_specs,
        out_specs=pl.BlockSpec((1, d, b), lambda i: (i + s0, 0, 0)),
        out_shape=jax.ShapeDtypeStruct((s_total, d, b), jnp.float32),
        **kwargs,
    )(*args)


_NCHUNK = 5


def kernel(indices, table):
    v, d = table.shape
    b, s = indices.shape
    scaled = _scale_table_t(table.T).reshape(2 * _HALF, d)
    idx = _idx_prep(indices.T.astype(jnp.int32)).reshape(1, b * s)
    wps = b // _WINDOW
    ns = s // _NCHUNK
    out_t = None
    for c in range(_NCHUNK):
        g = _sc_gather(scaled, idx, c * ns * wps, ns * wps)
        g2 = g.reshape(ns * b // 2, 2 * d)
        out_t = _transpose_fin(g2, s, c * ns, ns, b, prev=out_t)
    return out_t.transpose(2, 0, 1)
